# Initial kernel scaffold; baseline (speedup 1.0000x reference)
import jax
import jax.numpy as jnp
from jax.experimental import pallas as pl


def kernel(pts, grids):
    raise NotImplementedError("write your pallas kernel here")


# speedup vs baseline: 3.8680x; 3.8680x over previous
import functools

import numpy as np
import jax
import jax.numpy as jnp
from jax import lax
from jax.experimental import pallas as pl
from jax.experimental.pallas import tpu as pltpu
from jax.experimental.pallas import tpu_sc as plsc

MIN_RES = 16
MAX_RES = 512
NUM_LOD = 16
HASH_BANDWIDTH = 19
FEAT_DIM = 2
TABLE_SIZE = 2 ** HASH_BANDWIDTH
_b = np.exp((np.log(MAX_RES) - np.log(MIN_RES)) / (NUM_LOD - 1))
LODS = [int(1 + np.floor(MIN_RES * _b ** l)) for l in range(NUM_LOD)]
SIZES = [min(r ** 3, TABLE_SIZE) for r in LODS]
OFFS = np.concatenate([[0], np.cumsum(SIZES)]).astype(np.int32)
NUM_DENSE = sum(1 for r in LODS if r ** 3 <= TABLE_SIZE)

P1 = np.int32(2654435761 - 2 ** 32)
P2 = np.int32(805459861)
MASK = np.int32(TABLE_SIZE - 1)

N_PTS = 262144
NW = 32
CHUNK = 1024
NSTEP = CHUNK // 16
K = CHUNK // 128
NCHUNK = N_PTS // (NW * CHUNK)


def _body(xs_h, ys_h, zs_h, tab_h, res_h, off_h, out_h,
          xs_v, ys_v, zs_v, fx_v, fy_v, fz_v, idx_v, rows_v, out_v,
          res_v, off_v, sem):
    wid = lax.axis_index("s") * 2 + lax.axis_index("c")
    pltpu.sync_copy(res_h, res_v.at[pl.ds(0, 16)])
    pltpu.sync_copy(off_h, off_v.at[pl.ds(0, 16)])
    iota = lax.iota(jnp.int32, 16)
    zeros_i = jnp.zeros((16,), jnp.int32)
    ones_i = zeros_i + 1

    def chunk_body(ch, carry):
        base = wid * (NCHUNK * CHUNK) + ch * CHUNK
        pltpu.sync_copy(xs_h.at[pl.ds(base, CHUNK)], xs_v)
        pltpu.sync_copy(ys_h.at[pl.ds(base, CHUNK)], ys_v)
        pltpu.sync_copy(zs_h.at[pl.ds(base, CHUNK)], zs_v)

        def lod_body(l, dense):
            lvec = zeros_i + l
            res = plsc.load_gather(res_v, [lvec])
            off = plsc.load_gather(off_v, [lvec])
            scale = (res - 1).astype(jnp.float32)
            cap = res - 2
            res2 = res * res

            def idx_step(s, c2):
                p0 = s * 16
                x = xs_v[pl.ds(p0, 16)]
                y = ys_v[pl.ds(p0, 16)]
                z = zs_v[pl.ds(p0, 16)]
                sx = x * scale
                sy = y * scale
                sz = z * scale
                xi = jnp.minimum(sx.astype(jnp.int32), cap)
                yi = jnp.minimum(sy.astype(jnp.int32), cap)
                zi = jnp.minimum(sz.astype(jnp.int32), cap)
                fx_v[pl.ds(p0, 16)] = sx - xi.astype(jnp.float32)
                fy_v[pl.ds(p0, 16)] = sy - yi.astype(jnp.float32)
                fz_v[pl.ds(p0, 16)] = sz - zi.astype(jnp.float32)
                if dense:
                    ax = (xi, xi + 1)
                    ay = (yi * res, yi * res + res)
                    az = (zi * res2, zi * res2 + res2)
                else:
                    ax = (xi, xi + 1)
                    ay = (yi * P1, yi * P1 + P1)
                    az = (zi * P2, zi * P2 + P2)
                t0 = s >> 3
                o = (s & 7) * 16
                c = 0
                for dx in (0, 1):
                    for dy in (0, 1):
                        for dz in (0, 1):
                            if dense:
                                idx = ax[dx] + ay[dy] + az[dz]
                            else:
                                idx = (ax[dx] ^ ay[dy] ^ az[dz]) & MASK
                            idx_v[c * K + t0, pl.ds(o, 16)] = idx + off
                            c += 1
                return c2
            lax.fori_loop(0, NSTEP, idx_step, 0)

            def fire(t, c2):
                pltpu.make_async_copy(
                    tab_h.at[idx_v.at[t]],
                    rows_v.at[pl.ds(t * 128, 128)],
                    sem).start()
                return c2
            lax.fori_loop(0, 8 * K, fire, 0)

            def drain(t, c2):
                pltpu.make_async_copy(
                    tab_h.at[idx_v.at[t]],
                    rows_v.at[pl.ds(t * 128, 128)],
                    sem).wait()
                return c2
            lax.fori_loop(0, 8 * K, drain, 0)

            col0 = zeros_i + 2 * l
            col1 = col0 + 1

            def acc_step(s, c2):
                p0 = s * 16
                pvec = iota + p0
                fx = fx_v[pl.ds(p0, 16)]
                fy = fy_v[pl.ds(p0, 16)]
                fz = fz_v[pl.ds(p0, 16)]
                wx = (1.0 - fx, fx)
                wy = (1.0 - fy, fy)
                wz = (1.0 - fz, fz)
                acc0 = jnp.zeros((16,), jnp.float32)
                acc1 = jnp.zeros((16,), jnp.float32)
                c = 0
                for dx in (0, 1):
                    for dy in (0, 1):
                        for dz in (0, 1):
                            w = wx[dx] * wy[dy] * wz[dz]
                            rvec = pvec + c * CHUNK
                            g0 = plsc.load_gather(rows_v, [rvec, zeros_i])
                            g1 = plsc.load_gather(rows_v, [rvec, ones_i])
                            acc0 = acc0 + g0 * w
                            acc1 = acc1 + g1 * w
                            c += 1
                plsc.store_scatter(out_v, [pvec, col0], acc0)
                plsc.store_scatter(out_v, [pvec, col1], acc1)
                return c2
            lax.fori_loop(0, NSTEP, acc_step, 0)
            return 0

        lax.fori_loop(0, NUM_DENSE, lambda l, c: lod_body(l, True), 0)
        lax.fori_loop(NUM_DENSE, NUM_LOD, lambda l, c: lod_body(l, False), 0)
        pltpu.sync_copy(out_v, out_h.at[pl.ds(base, CHUNK)])
        return carry

    lax.fori_loop(0, NCHUNK, chunk_body, 0)


_mesh = plsc.VectorSubcoreMesh(core_axis_name="c", subcore_axis_name="s")

_hash_grid = pl.kernel(
    _body,
    out_type=jax.ShapeDtypeStruct((N_PTS, NUM_LOD * FEAT_DIM), jnp.float32),
    mesh=_mesh,
    compiler_params=pltpu.CompilerParams(
        needs_layout_passes=False, use_tc_tiling_on_sc=False),
    scratch_types=[
        pltpu.VMEM((CHUNK,), jnp.float32),
        pltpu.VMEM((CHUNK,), jnp.float32),
        pltpu.VMEM((CHUNK,), jnp.float32),
        pltpu.VMEM((CHUNK,), jnp.float32),
        pltpu.VMEM((CHUNK,), jnp.float32),
        pltpu.VMEM((CHUNK,), jnp.float32),
        pltpu.VMEM((8 * K, 128), jnp.int32),
        pltpu.VMEM((8 * CHUNK, FEAT_DIM), jnp.float32),
        pltpu.VMEM((CHUNK, NUM_LOD * FEAT_DIM), jnp.float32),
        pltpu.VMEM((128,), jnp.int32),
        pltpu.VMEM((128,), jnp.int32),
        pltpu.SemaphoreType.DMA,
    ],
)


def kernel(pts, grids):
    xs = pts[:, 0]
    ys = pts[:, 1]
    zs = pts[:, 2]
    table = jnp.concatenate(grids, axis=0)
    res_arr = jnp.asarray(LODS, dtype=jnp.int32)
    off_arr = jnp.asarray(OFFS[:NUM_LOD], dtype=jnp.int32)
    return _hash_grid(xs, ys, zs, table, res_arr, off_arr)


# speedup vs baseline: 6.2748x; 1.6222x over previous
import numpy as np
import jax
import jax.numpy as jnp
from jax import lax
from jax.experimental import pallas as pl
from jax.experimental.pallas import tpu as pltpu
from jax.experimental.pallas import tpu_sc as plsc

MIN_RES = 16
MAX_RES = 512
NUM_LOD = 16
HASH_BANDWIDTH = 19
FEAT_DIM = 2
TABLE_SIZE = 2 ** HASH_BANDWIDTH
_b = np.exp((np.log(MAX_RES) - np.log(MIN_RES)) / (NUM_LOD - 1))
LODS = [int(1 + np.floor(MIN_RES * _b ** l)) for l in range(NUM_LOD)]
DENSE = [r ** 3 <= TABLE_SIZE for r in LODS]

P1 = np.int32(2654435761 - 2 ** 32)
P2 = np.int32(805459861)
MASK = np.int32(TABLE_SIZE - 1)

N_PTS = 262144
NW = 32
CHUNK = 1024
NSTEP = CHUNK // 16
K = CHUNK // 128
NCHUNK = N_PTS // (NW * CHUNK)


def _body(xs_h, ys_h, zs_h, *refs):
    grid_hs = refs[:NUM_LOD]
    out_h = refs[NUM_LOD]
    (xs_v, ys_v, zs_v, fx_v, fy_v, fz_v, idx_v, rows_v, out_v, sem) = \
        refs[NUM_LOD + 1:]
    wid = lax.axis_index("s") * 2 + lax.axis_index("c")
    iota = lax.iota(jnp.int32, 16)
    zeros_i = jnp.zeros((16,), jnp.int32)
    ones_i = zeros_i + 1

    def chunk_body(ch, carry):
        base = wid * (NCHUNK * CHUNK) + ch * CHUNK
        pltpu.sync_copy(xs_h.at[pl.ds(base, CHUNK)], xs_v)
        pltpu.sync_copy(ys_h.at[pl.ds(base, CHUNK)], ys_v)
        pltpu.sync_copy(zs_h.at[pl.ds(base, CHUNK)], zs_v)

        for l in range(NUM_LOD):
            res = LODS[l]
            dense = DENSE[l]
            tab_h = grid_hs[l]
            scale = np.float32(res - 1)
            cap = np.int32(res - 2)
            res2 = np.int32(res * res)
            resi = np.int32(res)

            def idx_step(s, c2, dense=dense, scale=scale, cap=cap,
                         res2=res2, resi=resi):
                p0 = s * 16
                x = xs_v[pl.ds(p0, 16)]
                y = ys_v[pl.ds(p0, 16)]
                z = zs_v[pl.ds(p0, 16)]
                sx = x * scale
                sy = y * scale
                sz = z * scale
                xi = jnp.minimum(sx.astype(jnp.int32), cap)
                yi = jnp.minimum(sy.astype(jnp.int32), cap)
                zi = jnp.minimum(sz.astype(jnp.int32), cap)
                fx_v[pl.ds(p0, 16)] = sx - xi.astype(jnp.float32)
                fy_v[pl.ds(p0, 16)] = sy - yi.astype(jnp.float32)
                fz_v[pl.ds(p0, 16)] = sz - zi.astype(jnp.float32)
                if dense:
                    ax = (xi, xi + 1)
                    ay = (yi * resi, yi * resi + resi)
                    az = (zi * res2, zi * res2 + res2)
                else:
                    ax = (xi, xi + 1)
                    ay = (yi * P1, yi * P1 + P1)
                    az = (zi * P2, zi * P2 + P2)
                t0 = s >> 3
                o = (s & 7) * 16
                c = 0
                for dx in (0, 1):
                    for dy in (0, 1):
                        for dz in (0, 1):
                            if dense:
                                idx = ax[dx] + ay[dy] + az[dz]
                            else:
                                idx = (ax[dx] ^ ay[dy] ^ az[dz]) & MASK
                            idx_v[c * K + t0, pl.ds(o, 16)] = idx
                            c += 1
                return c2
            lax.fori_loop(0, NSTEP, idx_step, 0)

            def fire(t, c2, tab_h=tab_h):
                pltpu.make_async_copy(
                    tab_h.at[idx_v.at[t]],
                    rows_v.at[pl.ds(t * 128, 128)],
                    sem).start()
                return c2
            lax.fori_loop(0, 8 * K, fire, 0)

            def drain(t, c2, tab_h=tab_h):
                pltpu.make_async_copy(
                    tab_h.at[idx_v.at[t]],
                    rows_v.at[pl.ds(t * 128, 128)],
                    sem).wait()
                return c2
            lax.fori_loop(0, 8 * K, drain, 0)

            col0 = zeros_i + 2 * l
            col1 = col0 + 1

            def acc_step(s, c2, col0=col0, col1=col1):
                p0 = s * 16
                pvec = iota + p0
                fx = fx_v[pl.ds(p0, 16)]
                fy = fy_v[pl.ds(p0, 16)]
                fz = fz_v[pl.ds(p0, 16)]
                wx = (1.0 - fx, fx)
                wy = (1.0 - fy, fy)
                wz = (1.0 - fz, fz)
                acc0 = jnp.zeros((16,), jnp.float32)
                acc1 = jnp.zeros((16,), jnp.float32)
                c = 0
                for dx in (0, 1):
                    for dy in (0, 1):
                        for dz in (0, 1):
                            w = wx[dx] * wy[dy] * wz[dz]
                            rvec = pvec + c * CHUNK
                            g0 = plsc.load_gather(rows_v, [rvec, zeros_i])
                            g1 = plsc.load_gather(rows_v, [rvec, ones_i])
                            acc0 = acc0 + g0 * w
                            acc1 = acc1 + g1 * w
                            c += 1
                plsc.store_scatter(out_v, [pvec, col0], acc0)
                plsc.store_scatter(out_v, [pvec, col1], acc1)
                return c2
            lax.fori_loop(0, NSTEP, acc_step, 0)

        pltpu.sync_copy(out_v, out_h.at[pl.ds(base, CHUNK)])
        return carry

    lax.fori_loop(0, NCHUNK, chunk_body, 0)


_mesh = plsc.VectorSubcoreMesh(core_axis_name="c", subcore_axis_name="s")

_hash_grid = pl.kernel(
    _body,
    out_type=jax.ShapeDtypeStruct((N_PTS, NUM_LOD * FEAT_DIM), jnp.float32),
    mesh=_mesh,
    compiler_params=pltpu.CompilerParams(
        needs_layout_passes=False, use_tc_tiling_on_sc=False),
    scratch_types=[
        pltpu.VMEM((CHUNK,), jnp.float32),
        pltpu.VMEM((CHUNK,), jnp.float32),
        pltpu.VMEM((CHUNK,), jnp.float32),
        pltpu.VMEM((CHUNK,), jnp.float32),
        pltpu.VMEM((CHUNK,), jnp.float32),
        pltpu.VMEM((CHUNK,), jnp.float32),
        pltpu.VMEM((8 * K, 128), jnp.int32),
        pltpu.VMEM((8 * CHUNK, FEAT_DIM), jnp.float32),
        pltpu.VMEM((CHUNK, NUM_LOD * FEAT_DIM), jnp.float32),
        pltpu.SemaphoreType.DMA,
    ],
)


def kernel(pts, grids):
    xs = pts[:, 0]
    ys = pts[:, 1]
    zs = pts[:, 2]
    return _hash_grid(xs, ys, zs, *grids)


# speedup vs baseline: 12.0736x; 1.9241x over previous
import numpy as np
import jax
import jax.numpy as jnp
from jax import lax
from jax.experimental import pallas as pl
from jax.experimental.pallas import tpu as pltpu
from jax.experimental.pallas import tpu_sc as plsc

MIN_RES = 16
MAX_RES = 512
NUM_LOD = 16
HASH_BANDWIDTH = 19
FEAT_DIM = 2
TABLE_SIZE = 2 ** HASH_BANDWIDTH
_b = np.exp((np.log(MAX_RES) - np.log(MIN_RES)) / (NUM_LOD - 1))
LODS = [int(1 + np.floor(MIN_RES * _b ** l)) for l in range(NUM_LOD)]
SIZES = [min(r ** 3, TABLE_SIZE) for r in LODS]
DENSE = [r ** 3 <= TABLE_SIZE for r in LODS]

P1 = np.int32(2654435761 - 2 ** 32)
P2 = np.int32(805459861)
MASK = np.int32(TABLE_SIZE - 1)

N_PTS = 262144
NW = 32
CHUNK = 1024
NSTEP = CHUNK // 16
K = CHUNK // 128
NCHUNK = N_PTS // (NW * CHUNK)


def _body(*refs):
    xs_h, ys_h, zs_h = refs[0], refs[1], refs[2]
    col_hs = refs[3:3 + 2 * NUM_LOD]
    out_h = refs[3 + 2 * NUM_LOD]
    (xs_v, ys_v, zs_v, fx_v, fy_v, fz_v, idx_v, rowsa_v, rowsb_v,
     out_v, sem) = refs[4 + 2 * NUM_LOD:]
    wid = lax.axis_index("s") * 2 + lax.axis_index("c")
    iota = lax.iota(jnp.int32, 16)

    def chunk_body(ch, carry):
        base = wid * (NCHUNK * CHUNK) + ch * CHUNK
        pltpu.sync_copy(xs_h.at[pl.ds(base, CHUNK)], xs_v)
        pltpu.sync_copy(ys_h.at[pl.ds(base, CHUNK)], ys_v)
        pltpu.sync_copy(zs_h.at[pl.ds(base, CHUNK)], zs_v)

        for l in range(NUM_LOD):
            res = LODS[l]
            dense = DENSE[l]
            ca_h = col_hs[2 * l]
            cb_h = col_hs[2 * l + 1]
            scale = np.float32(res - 1)
            cap = np.int32(res - 2)
            res2 = np.int32(res * res)
            resi = np.int32(res)

            def idx_step(s, c2, dense=dense, scale=scale, cap=cap,
                         res2=res2, resi=resi):
                p0 = s * 16
                x = xs_v[pl.ds(p0, 16)]
                y = ys_v[pl.ds(p0, 16)]
                z = zs_v[pl.ds(p0, 16)]
                sx = x * scale
                sy = y * scale
                sz = z * scale
                xi = jnp.minimum(sx.astype(jnp.int32), cap)
                yi = jnp.minimum(sy.astype(jnp.int32), cap)
                zi = jnp.minimum(sz.astype(jnp.int32), cap)
                fx_v[pl.ds(p0, 16)] = sx - xi.astype(jnp.float32)
                fy_v[pl.ds(p0, 16)] = sy - yi.astype(jnp.float32)
                fz_v[pl.ds(p0, 16)] = sz - zi.astype(jnp.float32)
                if dense:
                    ax = (xi, xi + 1)
                    ay = (yi * resi, yi * resi + resi)
                    az = (zi * res2, zi * res2 + res2)
                else:
                    ax = (xi, xi + 1)
                    ay = (yi * P1, yi * P1 + P1)
                    az = (zi * P2, zi * P2 + P2)
                t0 = s >> 3
                o = (s & 7) * 16
                c = 0
                for dx in (0, 1):
                    for dy in (0, 1):
                        for dz in (0, 1):
                            if dense:
                                idx = ax[dx] + ay[dy] + az[dz]
                            else:
                                idx = (ax[dx] ^ ay[dy] ^ az[dz]) & MASK
                            idx_v[c * K + t0, pl.ds(o, 16)] = idx
                            c += 1
                return c2
            lax.fori_loop(0, NSTEP, idx_step, 0)

            def fire(t, c2, ca_h=ca_h, cb_h=cb_h):
                pltpu.make_async_copy(
                    ca_h.at[idx_v.at[t]],
                    rowsa_v.at[pl.ds(t * 128, 128)],
                    sem).start()
                pltpu.make_async_copy(
                    cb_h.at[idx_v.at[t]],
                    rowsb_v.at[pl.ds(t * 128, 128)],
                    sem).start()
                return c2
            lax.fori_loop(0, 8 * K, fire, 0)

            def drain(t, c2, ca_h=ca_h, cb_h=cb_h):
                pltpu.make_async_copy(
                    ca_h.at[idx_v.at[t]],
                    rowsa_v.at[pl.ds(t * 128, 128)],
                    sem).wait()
                pltpu.make_async_copy(
                    cb_h.at[idx_v.at[t]],
                    rowsb_v.at[pl.ds(t * 128, 128)],
                    sem).wait()
                return c2
            lax.fori_loop(0, 8 * K, drain, 0)

            col0 = jnp.zeros((16,), jnp.int32) + 2 * l
            col1 = col0 + 1

            def acc_step(s, c2, col0=col0, col1=col1):
                p0 = s * 16
                pvec = iota + p0
                fx = fx_v[pl.ds(p0, 16)]
                fy = fy_v[pl.ds(p0, 16)]
                fz = fz_v[pl.ds(p0, 16)]
                wx = (1.0 - fx, fx)
                wy = (1.0 - fy, fy)
                wz = (1.0 - fz, fz)
                acc0 = jnp.zeros((16,), jnp.float32)
                acc1 = jnp.zeros((16,), jnp.float32)
                c = 0
                for dx in (0, 1):
                    for dy in (0, 1):
                        for dz in (0, 1):
                            w = wx[dx] * wy[dy] * wz[dz]
                            rvec = pvec + c * CHUNK
                            g0 = plsc.load_gather(rowsa_v, [rvec])
                            g1 = plsc.load_gather(rowsb_v, [rvec])
                            acc0 = acc0 + g0 * w
                            acc1 = acc1 + g1 * w
                            c += 1
                plsc.store_scatter(out_v, [pvec, col0], acc0)
                plsc.store_scatter(out_v, [pvec, col1], acc1)
                return c2
            lax.fori_loop(0, NSTEP, acc_step, 0)

        pltpu.sync_copy(out_v, out_h.at[pl.ds(base, CHUNK)])
        return carry

    lax.fori_loop(0, NCHUNK, chunk_body, 0)


_mesh = plsc.VectorSubcoreMesh(core_axis_name="c", subcore_axis_name="s")

_hash_grid = pl.kernel(
    _body,
    out_type=jax.ShapeDtypeStruct((N_PTS, NUM_LOD * FEAT_DIM), jnp.float32),
    mesh=_mesh,
    compiler_params=pltpu.CompilerParams(
        needs_layout_passes=False, use_tc_tiling_on_sc=False),
    scratch_types=[
        pltpu.VMEM((CHUNK,), jnp.float32),
        pltpu.VMEM((CHUNK,), jnp.float32),
        pltpu.VMEM((CHUNK,), jnp.float32),
        pltpu.VMEM((CHUNK,), jnp.float32),
        pltpu.VMEM((CHUNK,), jnp.float32),
        pltpu.VMEM((CHUNK,), jnp.float32),
        pltpu.VMEM((8 * K, 128), jnp.int32),
        pltpu.VMEM((8 * CHUNK,), jnp.float32),
        pltpu.VMEM((8 * CHUNK,), jnp.float32),
        pltpu.VMEM((CHUNK, NUM_LOD * FEAT_DIM), jnp.float32),
        pltpu.SemaphoreType.DMA,
    ],
)


def kernel(pts, grids):
    xs = pts[:, 0]
    ys = pts[:, 1]
    zs = pts[:, 2]
    cols = []
    for g in grids:
        cols.append(g[:, 0])
        cols.append(g[:, 1])
    return _hash_grid(xs, ys, zs, *cols)


# speedup vs baseline: 15.5363x; 1.2868x over previous
import numpy as np
import jax
import jax.numpy as jnp
from jax import lax
from jax.experimental import pallas as pl
from jax.experimental.pallas import tpu as pltpu
from jax.experimental.pallas import tpu_sc as plsc

MIN_RES = 16
MAX_RES = 512
NUM_LOD = 16
HASH_BANDWIDTH = 19
FEAT_DIM = 2
TABLE_SIZE = 2 ** HASH_BANDWIDTH
_b = np.exp((np.log(MAX_RES) - np.log(MIN_RES)) / (NUM_LOD - 1))
LODS = [int(1 + np.floor(MIN_RES * _b ** l)) for l in range(NUM_LOD)]
SIZES = [min(r ** 3, TABLE_SIZE) for r in LODS]
DENSE = [r ** 3 <= TABLE_SIZE for r in LODS]
OFF_AL = []
_acc = 0
for _s in SIZES:
    OFF_AL.append(_acc)
    _acc += ((_s + 127) // 128) * 128
TOTAL_AL = _acc

P1 = np.int32(2654435761 - 2 ** 32)
P2 = np.int32(805459861)
MASK = np.int32(TABLE_SIZE - 1)

N_PTS = 262144
NW = 32
NS = 16
CHUNK = 512
NSTEP = CHUNK // 16
K = CHUNK // 128
NCHUNK = N_PTS // (NW * CHUNK)
RSPLIT = 2048


def _body(*refs):
    xs_h, ys_h, zs_h = refs[0], refs[1], refs[2]
    col_hs = refs[3:3 + 2 * NUM_LOD]
    out_h = refs[3 + 2 * NUM_LOD]
    pairs_h = refs[4 + 2 * NUM_LOD]
    (xs_v, ys_v, zs_v, fx_v, fy_v, fz_v, idx_v, rows_v,
     out_v, va_v, vb_v, st_v, sem) = refs[5 + 2 * NUM_LOD:]
    cid = lax.axis_index("c")
    sid = lax.axis_index("s")
    wid = sid * 2 + cid
    iota = lax.iota(jnp.int32, 16)
    zeros_i = jnp.zeros((16,), jnp.int32)
    ones_i = zeros_i + 1
    ptab = pairs_h.at[cid]

    for l in range(NUM_LOD):
        size = SIZES[l]
        off = OFF_AL[l]
        nch = (size + RSPLIT - 1) // RSPLIT
        nrounds = (nch + NS - 1) // NS
        last_a = max(0, ((size - RSPLIT) // 128) * 128)
        ca_h = col_hs[2 * l]
        cb_h = col_hs[2 * l + 1]

        def round_body(t, carry, ca_h=ca_h, cb_h=cb_h, off=off, nch=nch,
                       last_a=last_a):
            j = t * NS + sid

            @pl.when(j < nch)
            def _():
                a = jnp.where(j == nch - 1, last_a, j * RSPLIT)
                pltpu.sync_copy(ca_h.at[pl.ds(a, RSPLIT)], va_v)
                pltpu.sync_copy(cb_h.at[pl.ds(a, RSPLIT)], vb_v)

                def int_step(s, c2):
                    p0 = s * 16
                    iv = iota + p0
                    plsc.store_scatter(
                        st_v, [iv, zeros_i], va_v[pl.ds(p0, 16)])
                    plsc.store_scatter(
                        st_v, [iv, ones_i], vb_v[pl.ds(p0, 16)])
                    return c2
                lax.fori_loop(0, RSPLIT // 16, int_step, 0)
                pltpu.sync_copy(st_v, ptab.at[pl.ds(off + a, RSPLIT)])
            return carry

        lax.fori_loop(0, nrounds, round_body, 0)

    plsc.subcore_barrier()

    def chunk_body(ch, carry):
        base = wid * (NCHUNK * CHUNK) + ch * CHUNK
        pltpu.sync_copy(xs_h.at[pl.ds(base, CHUNK)], xs_v)
        pltpu.sync_copy(ys_h.at[pl.ds(base, CHUNK)], ys_v)
        pltpu.sync_copy(zs_h.at[pl.ds(base, CHUNK)], zs_v)

        for l in range(NUM_LOD):
            res = LODS[l]
            dense = DENSE[l]
            off = np.int32(OFF_AL[l])
            scale = np.float32(res - 1)
            cap = np.int32(res - 2)
            res2 = np.int32(res * res)
            resi = np.int32(res)

            def idx_step(s, c2, dense=dense, scale=scale, cap=cap,
                         res2=res2, resi=resi, off=off):
                p0 = s * 16
                x = xs_v[pl.ds(p0, 16)]
                y = ys_v[pl.ds(p0, 16)]
                z = zs_v[pl.ds(p0, 16)]
                sx = x * scale
                sy = y * scale
                sz = z * scale
                xi = jnp.minimum(sx.astype(jnp.int32), cap)
                yi = jnp.minimum(sy.astype(jnp.int32), cap)
                zi = jnp.minimum(sz.astype(jnp.int32), cap)
                fx_v[pl.ds(p0, 16)] = sx - xi.astype(jnp.float32)
                fy_v[pl.ds(p0, 16)] = sy - yi.astype(jnp.float32)
                fz_v[pl.ds(p0, 16)] = sz - zi.astype(jnp.float32)
                if dense:
                    ax = (xi + off, xi + (off + 1))
                    ay = (yi * resi, yi * resi + resi)
                    az = (zi * res2, zi * res2 + res2)
                else:
                    ax = (xi, xi + 1)
                    ay = (yi * P1, yi * P1 + P1)
                    az = (zi * P2, zi * P2 + P2)
                t0 = s >> 3
                o = (s & 7) * 16
                c = 0
                for dx in (0, 1):
                    for dy in (0, 1):
                        for dz in (0, 1):
                            if dense:
                                idx = ax[dx] + ay[dy] + az[dz]
                            else:
                                idx = ((ax[dx] ^ ay[dy] ^ az[dz]) & MASK) + off
                            idx_v[c * K + t0, pl.ds(o, 16)] = idx
                            c += 1
                return c2
            lax.fori_loop(0, NSTEP, idx_step, 0)

            def fire(t, c2):
                pltpu.make_async_copy(
                    ptab.at[idx_v.at[t]],
                    rows_v.at[pl.ds(t * 128, 128)],
                    sem).start()
                return c2
            lax.fori_loop(0, 8 * K, fire, 0)

            def drain(t, c2):
                pltpu.make_async_copy(
                    ptab.at[idx_v.at[t]],
                    rows_v.at[pl.ds(t * 128, 128)],
                    sem).wait()
                return c2
            lax.fori_loop(0, 8 * K, drain, 0)

            col0 = zeros_i + 2 * l
            col1 = col0 + 1

            def acc_step(s, c2, col0=col0, col1=col1):
                p0 = s * 16
                pvec = iota + p0
                fx = fx_v[pl.ds(p0, 16)]
                fy = fy_v[pl.ds(p0, 16)]
                fz = fz_v[pl.ds(p0, 16)]
                wx = (1.0 - fx, fx)
                wy = (1.0 - fy, fy)
                wz = (1.0 - fz, fz)
                acc0 = jnp.zeros((16,), jnp.float32)
                acc1 = jnp.zeros((16,), jnp.float32)
                c = 0
                for dx in (0, 1):
                    for dy in (0, 1):
                        for dz in (0, 1):
                            w = wx[dx] * wy[dy] * wz[dz]
                            rvec = pvec + c * CHUNK
                            g0 = plsc.load_gather(rows_v, [rvec, zeros_i])
                            g1 = plsc.load_gather(rows_v, [rvec, ones_i])
                            acc0 = acc0 + g0 * w
                            acc1 = acc1 + g1 * w
                            c += 1
                plsc.store_scatter(out_v, [pvec, col0], acc0)
                plsc.store_scatter(out_v, [pvec, col1], acc1)
                return c2
            lax.fori_loop(0, NSTEP, acc_step, 0)

        pltpu.sync_copy(out_v, out_h.at[pl.ds(base, CHUNK)])
        return carry

    lax.fori_loop(0, NCHUNK, chunk_body, 0)


_mesh = plsc.VectorSubcoreMesh(core_axis_name="c", subcore_axis_name="s")

_hash_grid = pl.kernel(
    _body,
    out_type=(
        jax.ShapeDtypeStruct((N_PTS, NUM_LOD * FEAT_DIM), jnp.float32),
        jax.ShapeDtypeStruct((2, TOTAL_AL, FEAT_DIM), jnp.float32),
    ),
    mesh=_mesh,
    compiler_params=pltpu.CompilerParams(
        needs_layout_passes=False, use_tc_tiling_on_sc=False),
    scratch_types=[
        pltpu.VMEM((CHUNK,), jnp.float32),
        pltpu.VMEM((CHUNK,), jnp.float32),
        pltpu.VMEM((CHUNK,), jnp.float32),
        pltpu.VMEM((CHUNK,), jnp.float32),
        pltpu.VMEM((CHUNK,), jnp.float32),
        pltpu.VMEM((CHUNK,), jnp.float32),
        pltpu.VMEM((8 * K, 128), jnp.int32),
        pltpu.VMEM((8 * CHUNK, FEAT_DIM), jnp.float32),
        pltpu.VMEM((CHUNK, NUM_LOD * FEAT_DIM), jnp.float32),
        pltpu.VMEM((RSPLIT,), jnp.float32),
        pltpu.VMEM((RSPLIT,), jnp.float32),
        pltpu.VMEM((RSPLIT, FEAT_DIM), jnp.float32),
        pltpu.SemaphoreType.DMA,
    ],
)


def kernel(pts, grids):
    xs = pts[:, 0]
    ys = pts[:, 1]
    zs = pts[:, 2]
    cols = []
    for g in grids:
        cols.append(g[:, 0])
        cols.append(g[:, 1])
    out, _pairs = _hash_grid(xs, ys, zs, *cols)
    return out
